# baseline (speedup 1.0000x reference)
import jax
import jax.numpy as jnp
from jax import lax
from jax.experimental import pallas as pl
from jax.experimental.pallas import tpu as pltpu
from jax.experimental.pallas import tpu_sc as plsc

D = 128
E_TOTAL = 640000
CHUNK = 128
NCHUNKS = E_TOTAL // CHUNK
NPOS_CHUNKS = 320000 // CHUNK
NC, NS = 2, 16
NW = NC * NS
ITERS = -(-NCHUNKS // NW)
NCHUNKS_PAD = NW * ITERS
E_PAD = NCHUNKS_PAD * CHUNK
HBITS = 18
NBUCK = 1 << HBITS
HISTLEN = 2 * NBUCK
STRIPE = HISTLEN // NS


def _sc_body(emb, srcs, dsts, zeros, scores_out, hist_out,
             idx_u0, idx_v0, idx_u1, idx_v1,
             rows_u0, rows_v0, rows_u1, rows_v1,
             sbuf, bbuf, ones, hist_sh,
             sem_i0, sem_i1, sem_gu0, sem_gv0, sem_gu1, sem_gv1):
    cid = lax.axis_index("c")
    sid = lax.axis_index("s")
    wid = sid * NC + cid

    idx_u = [idx_u0, idx_u1]
    idx_v = [idx_v0, idx_v1]
    rows_u = [rows_u0, rows_u1]
    rows_v = [rows_v0, rows_v1]
    sem_i = [sem_i0, sem_i1]
    sem_gu = [sem_gu0, sem_gu1]
    sem_gv = [sem_gv0, sem_gv1]

    pltpu.sync_copy(zeros, hist_sh.at[pl.ds(sid * STRIPE, STRIPE)])
    for j in range(CHUNK // 16):
        ones[pl.ds(j * 16, 16)] = jnp.full((16,), 1.0, jnp.float32)
    plsc.subcore_barrier()

    def cix(i):
        return wid + NW * i

    def fire_idx(i, s):
        base = cix(i) * CHUNK
        pltpu.async_copy(srcs.at[pl.ds(base, CHUNK)], idx_u[s], sem_i[s])
        pltpu.async_copy(dsts.at[pl.ds(base, CHUNK)], idx_v[s], sem_i[s])

    def wait_idx(s):
        pltpu.make_async_copy(srcs.at[pl.ds(0, CHUNK)], idx_u[s], sem_i[s]).wait()
        pltpu.make_async_copy(dsts.at[pl.ds(0, CHUNK)], idx_v[s], sem_i[s]).wait()

    def fire_gather(s):
        pltpu.async_copy(emb.at[idx_u[s]], rows_u[s], sem_gu[s])
        pltpu.async_copy(emb.at[idx_v[s]], rows_v[s], sem_gv[s])

    def wait_gather(s):
        pltpu.make_async_copy(emb.at[idx_u[s]], rows_u[s], sem_gu[s]).wait()
        pltpu.make_async_copy(emb.at[idx_v[s]], rows_v[s], sem_gv[s]).wait()

    lane = lax.iota(jnp.int32, 16)

    def compute(i, s):
        c = cix(i)
        ru, rv = rows_u[s], rows_v[s]
        def group_body(g, carry2):
            rows16 = g * 16 + lane

            z = jnp.zeros((16,), jnp.float32)

            @plsc.parallel_loop(0, D // 4, unroll=8, carry=(z, z, z, z))
            def accs(db, accs_in):
                new = []
                for k in range(4):
                    col = jnp.full((16,), db * 4 + k, jnp.int32)
                    cu = plsc.load_gather(ru, [rows16, col])
                    cv = plsc.load_gather(rv, [rows16, col])
                    new.append(accs_in[k] + cu * cv)
                return tuple(new)
            vec = (accs[0] + accs[1]) + (accs[2] + accs[3])
            sbuf[pl.ds(g * 16, 16)] = vec
            return carry2

        lax.fori_loop(0, CHUNK // 16, group_body, 0)

        off = (jnp.where(c < NPOS_CHUNKS, 0, NBUCK)
               + jnp.where(c < NCHUNKS, 0, NBUCK)).astype(jnp.int32)
        for j in range(CHUNK // 16):
            sv = sbuf[pl.ds(j * 16, 16)]
            b = plsc.bitcast(sv, jnp.int32)
            key = jnp.where(b < 0, b ^ jnp.int32(0x7FFFFFFF), b)
            bkt = lax.shift_right_arithmetic(key, 32 - HBITS)
            bkt = bkt + jnp.int32(NBUCK // 2) + off
            bbuf[pl.ds(j * 16, 16)] = bkt

        pltpu.sync_copy(ones, hist_sh.at[bbuf], add=True)
        pltpu.sync_copy(sbuf, scores_out.at[pl.ds(c * CHUNK, CHUNK)])

    fire_idx(0, 0)
    fire_idx(1, 1)
    wait_idx(0)
    fire_gather(0)

    def loop_body(j, carry):
        wait_gather(0)

        @pl.when(2 * j + 2 <= ITERS - 1)
        def _():
            fire_idx(2 * j + 2, 0)

        @pl.when(2 * j + 1 <= ITERS - 1)
        def _():
            wait_idx(1)
            fire_gather(1)

        compute(2 * j, 0)

        @pl.when(2 * j + 1 <= ITERS - 1)
        def _():
            wait_gather(1)

            @pl.when(2 * j + 3 <= ITERS - 1)
            def _():
                fire_idx(2 * j + 3, 1)

            @pl.when(2 * j + 2 <= ITERS - 1)
            def _():
                wait_idx(0)
                fire_gather(0)

            compute(2 * j + 1, 1)

        return carry

    lax.fori_loop(0, (ITERS + 1) // 2, loop_body, 0)

    plsc.subcore_barrier()
    pltpu.sync_copy(hist_sh.at[pl.ds(sid * STRIPE, STRIPE)],
                    hist_out.at[cid, pl.ds(sid * STRIPE, STRIPE)])


_SC_SCORE_CACHE = []


def _sc_score_fn():
    if not _SC_SCORE_CACHE:
        _SC_SCORE_CACHE.append(_build_sc_score())
    return _SC_SCORE_CACHE[0]


def _build_sc_score():
  return pl.kernel(
    _sc_body,
    out_type=(
        jax.ShapeDtypeStruct((E_PAD,), jnp.float32),
        jax.ShapeDtypeStruct((NC, HISTLEN), jnp.float32),
    ),
    mesh=plsc.VectorSubcoreMesh(core_axis_name="c", subcore_axis_name="s",
                                num_cores=NC, num_subcores=NS),
    compiler_params=pltpu.CompilerParams(needs_layout_passes=False),
    scratch_types=[
        pltpu.VMEM((CHUNK,), jnp.int32),
        pltpu.VMEM((CHUNK,), jnp.int32),
        pltpu.VMEM((CHUNK,), jnp.int32),
        pltpu.VMEM((CHUNK,), jnp.int32),
        pltpu.VMEM((CHUNK, D), jnp.float32),
        pltpu.VMEM((CHUNK, D), jnp.float32),
        pltpu.VMEM((CHUNK, D), jnp.float32),
        pltpu.VMEM((CHUNK, D), jnp.float32),
        pltpu.VMEM((CHUNK,), jnp.float32),
        pltpu.VMEM((CHUNK,), jnp.int32),
        pltpu.VMEM((CHUNK,), jnp.float32),
        pltpu.VMEM_SHARED((HISTLEN + NBUCK,), jnp.float32),
        pltpu.SemaphoreType.DMA,
        pltpu.SemaphoreType.DMA,
        pltpu.SemaphoreType.DMA,
        pltpu.SemaphoreType.DMA,
        pltpu.SemaphoreType.DMA,
        pltpu.SemaphoreType.DMA,
    ],
  )


R = 512
C = NBUCK // R


def _tc_body(scores_ref, hp_ref, hn_ref, out_ref):
    s = scores_ref[...]
    rows = lax.broadcasted_iota(jnp.int32, s.shape, 0)
    lbl = (rows < NPOS_CHUNKS).astype(jnp.float32)
    bce = (jnp.maximum(s, 0.0) - s * lbl
           + jnp.log1p(jnp.exp(-jnp.abs(s))))
    loss = jnp.sum(bce) * (1.0 / E_TOTAL)

    p = hp_ref[0] + hp_ref[1]
    n = hn_ref[0] + hn_ref[1]
    ii = lax.broadcasted_iota(jnp.int32, (R, C), 0)
    jj = lax.broadcasted_iota(jnp.int32, (R, C), 1)
    upper = (ii < jj).astype(jnp.float32)
    lower = (ii > jj).astype(jnp.float32)
    cum_in_row = jnp.dot(n, upper, preferred_element_type=jnp.float32)
    rowsum = jnp.sum(n, axis=1, keepdims=True)
    rowprev = jnp.dot(lower, rowsum, preferred_element_type=jnp.float32)
    cum_excl = cum_in_row + rowprev

    inv = 1.0 / 320000.0
    term1 = jnp.sum(p * (cum_excl * inv))
    term2 = 0.5 * inv * jnp.sum(p * n)
    auc = (term1 + term2) * inv

    ij = lax.broadcasted_iota(jnp.int32, (8, 128), 1)
    i0 = lax.broadcasted_iota(jnp.int32, (8, 128), 0)
    out = jnp.where((i0 == 0) & (ij == 0), loss,
                    jnp.where((i0 == 0) & (ij == 1), auc, 0.0))
    out_ref[...] = out


def kernel(embeddings, pos_edges, neg_edges):
    pad = jnp.zeros((E_PAD - E_TOTAL,), jnp.int32)
    srcs = jnp.concatenate(
        [pos_edges[0].astype(jnp.int32), neg_edges[0].astype(jnp.int32), pad])
    dsts = jnp.concatenate(
        [pos_edges[1].astype(jnp.int32), neg_edges[1].astype(jnp.int32), pad])
    zeros = jnp.zeros((STRIPE,), jnp.float32)
    scores, hist = _sc_score_fn()(embeddings, srcs, dsts, zeros)

    scores2d = lax.slice(scores, (0,), (E_TOTAL,)).reshape(NCHUNKS, CHUNK)
    hp = hist[:, :NBUCK].reshape(NC, R, C)
    hn = hist[:, NBUCK:].reshape(NC, R, C)
    out = pl.pallas_call(
        _tc_body,
        out_shape=jax.ShapeDtypeStruct((8, 128), jnp.float32),
    )(scores2d, hp, hn)
    return out[0, 0], out[0, 1]


# speedup vs baseline: 4.1844x; 4.1844x over previous
import jax
import jax.numpy as jnp
from jax import lax
from jax.experimental import pallas as pl
from jax.experimental.pallas import tpu as pltpu
from jax.experimental.pallas import tpu_sc as plsc

D = 128
E_TOTAL = 640000
CHUNK = 128
NCHUNKS = E_TOTAL // CHUNK
NPOS_CHUNKS = 320000 // CHUNK
NC, NS = 2, 16
NW = NC * NS
ITERS = -(-NCHUNKS // NW)
NCHUNKS_PAD = NW * ITERS
E_PAD = NCHUNKS_PAD * CHUNK
HBITS = 18
NBUCK = 1 << HBITS
HISTLEN = 2 * NBUCK
STRIPE = HISTLEN // NS


def _sc_body(emb, srcs, dsts, zeros, scores_out, hist_out,
             idx_u0, idx_v0, idx_u1, idx_v1,
             rows_u0, rows_v0, rows_u1, rows_v1,
             sbuf, bbuf, ones, hist_sh,
             sem_i0, sem_i1, sem_gu0, sem_gv0, sem_gu1, sem_gv1):
    cid = lax.axis_index("c")
    sid = lax.axis_index("s")
    wid = sid * NC + cid

    idx_u = [idx_u0, idx_u1]
    idx_v = [idx_v0, idx_v1]
    rows_u = [rows_u0, rows_u1]
    rows_v = [rows_v0, rows_v1]
    sem_i = [sem_i0, sem_i1]
    sem_gu = [sem_gu0, sem_gu1]
    sem_gv = [sem_gv0, sem_gv1]

    pltpu.sync_copy(zeros, hist_sh.at[pl.ds(sid * STRIPE, STRIPE)])
    for j in range(CHUNK // 16):
        ones[pl.ds(j * 16, 16)] = jnp.full((16,), 1.0, jnp.float32)
    plsc.subcore_barrier()

    def cix(i):
        return wid + NW * i

    def fire_idx(i, s):
        base = cix(i) * CHUNK
        pltpu.async_copy(srcs.at[pl.ds(base, CHUNK)], idx_u[s], sem_i[s])
        pltpu.async_copy(dsts.at[pl.ds(base, CHUNK)], idx_v[s], sem_i[s])

    def wait_idx(s):
        pltpu.make_async_copy(srcs.at[pl.ds(0, CHUNK)], idx_u[s], sem_i[s]).wait()
        pltpu.make_async_copy(dsts.at[pl.ds(0, CHUNK)], idx_v[s], sem_i[s]).wait()

    def fire_gather(s):
        pltpu.async_copy(emb.at[idx_u[s]], rows_u[s], sem_gu[s])
        pltpu.async_copy(emb.at[idx_v[s]], rows_v[s], sem_gv[s])

    def wait_gather(s):
        pltpu.make_async_copy(emb.at[idx_u[s]], rows_u[s], sem_gu[s]).wait()
        pltpu.make_async_copy(emb.at[idx_v[s]], rows_v[s], sem_gv[s]).wait()

    lane = lax.iota(jnp.int32, 16)

    def compute(i, s):
        c = cix(i)
        ru, rv = rows_u[s], rows_v[s]
        def group_body(g, carry2):
            rows16 = g * 16 + lane

            z = jnp.zeros((16,), jnp.float32)

            @plsc.parallel_loop(0, D // 4, unroll=8, carry=(z, z, z, z))
            def accs(db, accs_in):
                new = []
                for k in range(4):
                    col = (lane + (db * 4 + k)) & jnp.int32(D - 1)
                    cu = plsc.load_gather(ru, [rows16, col])
                    cv = plsc.load_gather(rv, [rows16, col])
                    new.append(accs_in[k] + cu * cv)
                return tuple(new)
            vec = (accs[0] + accs[1]) + (accs[2] + accs[3])
            sbuf[pl.ds(g * 16, 16)] = vec
            return carry2

        lax.fori_loop(0, CHUNK // 16, group_body, 0)

        off = (jnp.where(c < NPOS_CHUNKS, 0, NBUCK)
               + jnp.where(c < NCHUNKS, 0, NBUCK)).astype(jnp.int32)
        for j in range(CHUNK // 16):
            sv = sbuf[pl.ds(j * 16, 16)]
            b = plsc.bitcast(sv, jnp.int32)
            key = jnp.where(b < 0, b ^ jnp.int32(0x7FFFFFFF), b)
            bkt = lax.shift_right_arithmetic(key, 32 - HBITS)
            bkt = bkt + jnp.int32(NBUCK // 2) + off
            bbuf[pl.ds(j * 16, 16)] = bkt

        pltpu.sync_copy(ones, hist_sh.at[bbuf], add=True)
        pltpu.sync_copy(sbuf, scores_out.at[pl.ds(c * CHUNK, CHUNK)])

    fire_idx(0, 0)
    fire_idx(1, 1)
    wait_idx(0)
    fire_gather(0)

    def loop_body(j, carry):
        wait_gather(0)

        @pl.when(2 * j + 2 <= ITERS - 1)
        def _():
            fire_idx(2 * j + 2, 0)

        @pl.when(2 * j + 1 <= ITERS - 1)
        def _():
            wait_idx(1)
            fire_gather(1)

        compute(2 * j, 0)

        @pl.when(2 * j + 1 <= ITERS - 1)
        def _():
            wait_gather(1)

            @pl.when(2 * j + 3 <= ITERS - 1)
            def _():
                fire_idx(2 * j + 3, 1)

            @pl.when(2 * j + 2 <= ITERS - 1)
            def _():
                wait_idx(0)
                fire_gather(0)

            compute(2 * j + 1, 1)

        return carry

    lax.fori_loop(0, (ITERS + 1) // 2, loop_body, 0)

    plsc.subcore_barrier()
    pltpu.sync_copy(hist_sh.at[pl.ds(sid * STRIPE, STRIPE)],
                    hist_out.at[cid, pl.ds(sid * STRIPE, STRIPE)])


_SC_SCORE_CACHE = []


def _sc_score_fn():
    if not _SC_SCORE_CACHE:
        _SC_SCORE_CACHE.append(_build_sc_score())
    return _SC_SCORE_CACHE[0]


def _build_sc_score():
  return pl.kernel(
    _sc_body,
    out_type=(
        jax.ShapeDtypeStruct((E_PAD,), jnp.float32),
        jax.ShapeDtypeStruct((NC, HISTLEN), jnp.float32),
    ),
    mesh=plsc.VectorSubcoreMesh(core_axis_name="c", subcore_axis_name="s",
                                num_cores=NC, num_subcores=NS),
    compiler_params=pltpu.CompilerParams(needs_layout_passes=False),
    scratch_types=[
        pltpu.VMEM((CHUNK,), jnp.int32),
        pltpu.VMEM((CHUNK,), jnp.int32),
        pltpu.VMEM((CHUNK,), jnp.int32),
        pltpu.VMEM((CHUNK,), jnp.int32),
        pltpu.VMEM((CHUNK, D), jnp.float32),
        pltpu.VMEM((CHUNK, D), jnp.float32),
        pltpu.VMEM((CHUNK, D), jnp.float32),
        pltpu.VMEM((CHUNK, D), jnp.float32),
        pltpu.VMEM((CHUNK,), jnp.float32),
        pltpu.VMEM((CHUNK,), jnp.int32),
        pltpu.VMEM((CHUNK,), jnp.float32),
        pltpu.VMEM_SHARED((HISTLEN + NBUCK,), jnp.float32),
        pltpu.SemaphoreType.DMA,
        pltpu.SemaphoreType.DMA,
        pltpu.SemaphoreType.DMA,
        pltpu.SemaphoreType.DMA,
        pltpu.SemaphoreType.DMA,
        pltpu.SemaphoreType.DMA,
    ],
  )


R = 512
C = NBUCK // R


def _tc_body(scores_ref, hp_ref, hn_ref, out_ref):
    s = scores_ref[...]
    rows = lax.broadcasted_iota(jnp.int32, s.shape, 0)
    lbl = (rows < NPOS_CHUNKS).astype(jnp.float32)
    bce = (jnp.maximum(s, 0.0) - s * lbl
           + jnp.log1p(jnp.exp(-jnp.abs(s))))
    loss = jnp.sum(bce) * (1.0 / E_TOTAL)

    p = hp_ref[0] + hp_ref[1]
    n = hn_ref[0] + hn_ref[1]
    ii = lax.broadcasted_iota(jnp.int32, (R, C), 0)
    jj = lax.broadcasted_iota(jnp.int32, (R, C), 1)
    upper = (ii < jj).astype(jnp.float32)
    lower = (ii > jj).astype(jnp.float32)
    cum_in_row = jnp.dot(n, upper, preferred_element_type=jnp.float32)
    rowsum = jnp.sum(n, axis=1, keepdims=True)
    rowprev = jnp.dot(lower, rowsum, preferred_element_type=jnp.float32)
    cum_excl = cum_in_row + rowprev

    inv = 1.0 / 320000.0
    term1 = jnp.sum(p * (cum_excl * inv))
    term2 = 0.5 * inv * jnp.sum(p * n)
    auc = (term1 + term2) * inv

    ij = lax.broadcasted_iota(jnp.int32, (8, 128), 1)
    i0 = lax.broadcasted_iota(jnp.int32, (8, 128), 0)
    out = jnp.where((i0 == 0) & (ij == 0), loss,
                    jnp.where((i0 == 0) & (ij == 1), auc, 0.0))
    out_ref[...] = out


def kernel(embeddings, pos_edges, neg_edges):
    pad = jnp.zeros((E_PAD - E_TOTAL,), jnp.int32)
    srcs = jnp.concatenate(
        [pos_edges[0].astype(jnp.int32), neg_edges[0].astype(jnp.int32), pad])
    dsts = jnp.concatenate(
        [pos_edges[1].astype(jnp.int32), neg_edges[1].astype(jnp.int32), pad])
    zeros = jnp.zeros((STRIPE,), jnp.float32)
    scores, hist = _sc_score_fn()(embeddings, srcs, dsts, zeros)

    scores2d = lax.slice(scores, (0,), (E_TOTAL,)).reshape(NCHUNKS, CHUNK)
    hp = hist[:, :NBUCK].reshape(NC, R, C)
    hn = hist[:, NBUCK:].reshape(NC, R, C)
    out = pl.pallas_call(
        _tc_body,
        out_shape=jax.ShapeDtypeStruct((8, 128), jnp.float32),
    )(scores2d, hp, hn)
    return out[0, 0], out[0, 1]


# speedup vs baseline: 4.2079x; 1.0056x over previous
import jax
import jax.numpy as jnp
from jax import lax
from jax.experimental import pallas as pl
from jax.experimental.pallas import tpu as pltpu
from jax.experimental.pallas import tpu_sc as plsc

D = 128
E_TOTAL = 640000
CHUNK = 128
NCHUNKS = E_TOTAL // CHUNK
NPOS_CHUNKS = 320000 // CHUNK
NC, NS = 2, 16
NW = NC * NS
ITERS = -(-NCHUNKS // NW)
NCHUNKS_PAD = NW * ITERS
E_PAD = NCHUNKS_PAD * CHUNK
HBITS = 18
NBUCK = 1 << HBITS
HISTLEN = 2 * NBUCK
STRIPE = HISTLEN // NS


def _sc_body(emb, srcs, dsts, zeros, scores_out, hist_out,
             idx_u0, idx_v0, idx_u1, idx_v1,
             rows_u0, rows_v0, rows_u1, rows_v1,
             sbuf0, sbuf1, bbuf0, bbuf1, ones, hist_sh,
             sem_i0, sem_i1, sem_gu0, sem_gv0, sem_gu1, sem_gv1,
             sem_sc0, sem_sc1, sem_sw0, sem_sw1):
    cid = lax.axis_index("c")
    sid = lax.axis_index("s")
    wid = sid * NC + cid

    idx_u = [idx_u0, idx_u1]
    idx_v = [idx_v0, idx_v1]
    rows_u = [rows_u0, rows_u1]
    rows_v = [rows_v0, rows_v1]
    sbuf_r = [sbuf0, sbuf1]
    bbuf_r = [bbuf0, bbuf1]
    sem_i = [sem_i0, sem_i1]
    sem_gu = [sem_gu0, sem_gu1]
    sem_gv = [sem_gv0, sem_gv1]
    sem_sc = [sem_sc0, sem_sc1]
    sem_sw = [sem_sw0, sem_sw1]

    pltpu.sync_copy(zeros, hist_sh.at[pl.ds(sid * STRIPE, STRIPE)])
    for j in range(CHUNK // 16):
        ones[pl.ds(j * 16, 16)] = jnp.full((16,), 1.0, jnp.float32)
    plsc.subcore_barrier()

    def cix(i):
        return wid + NW * i

    def fire_idx(i, s):
        base = cix(i) * CHUNK
        pltpu.async_copy(srcs.at[pl.ds(base, CHUNK)], idx_u[s], sem_i[s])
        pltpu.async_copy(dsts.at[pl.ds(base, CHUNK)], idx_v[s], sem_i[s])

    def wait_idx(s):
        pltpu.make_async_copy(srcs.at[pl.ds(0, CHUNK)], idx_u[s], sem_i[s]).wait()
        pltpu.make_async_copy(dsts.at[pl.ds(0, CHUNK)], idx_v[s], sem_i[s]).wait()

    def fire_gather(s):
        pltpu.async_copy(emb.at[idx_u[s]], rows_u[s], sem_gu[s])
        pltpu.async_copy(emb.at[idx_v[s]], rows_v[s], sem_gv[s])

    def wait_gather(s):
        pltpu.make_async_copy(emb.at[idx_u[s]], rows_u[s], sem_gu[s]).wait()
        pltpu.make_async_copy(emb.at[idx_v[s]], rows_v[s], sem_gv[s]).wait()

    lane = lax.iota(jnp.int32, 16)

    def compute(i, s):
        c = cix(i)
        ru, rv = rows_u[s], rows_v[s]
        sbuf, bbuf = sbuf_r[s], bbuf_r[s]

        @pl.when(i >= 2)
        def _():
            pltpu.make_async_copy(ones, hist_sh.at[bbuf], sem_sc[s]).wait()
            pltpu.make_async_copy(
                sbuf, scores_out.at[pl.ds(0, CHUNK)], sem_sw[s]).wait()
        def group_body(g, carry2):
            rows16 = g * 16 + lane

            z = jnp.zeros((16,), jnp.float32)

            @plsc.parallel_loop(0, D // 4, unroll=8, carry=(z, z, z, z))
            def accs(db, accs_in):
                new = []
                for k in range(4):
                    col = (lane + (db * 4 + k)) & jnp.int32(D - 1)
                    cu = plsc.load_gather(ru, [rows16, col])
                    cv = plsc.load_gather(rv, [rows16, col])
                    new.append(accs_in[k] + cu * cv)
                return tuple(new)
            vec = (accs[0] + accs[1]) + (accs[2] + accs[3])
            sbuf[pl.ds(g * 16, 16)] = vec
            return carry2

        lax.fori_loop(0, CHUNK // 16, group_body, 0)

        off = (jnp.where(c < NPOS_CHUNKS, 0, NBUCK)
               + jnp.where(c < NCHUNKS, 0, NBUCK)).astype(jnp.int32)
        for j in range(CHUNK // 16):
            sv = sbuf[pl.ds(j * 16, 16)]
            b = plsc.bitcast(sv, jnp.int32)
            key = jnp.where(b < 0, b ^ jnp.int32(0x7FFFFFFF), b)
            bkt = lax.shift_right_arithmetic(key, 32 - HBITS)
            bkt = bkt + jnp.int32(NBUCK // 2) + off
            bbuf[pl.ds(j * 16, 16)] = bkt

        pltpu.async_copy(ones, hist_sh.at[bbuf], sem_sc[s], add=True)
        pltpu.async_copy(sbuf, scores_out.at[pl.ds(c * CHUNK, CHUNK)],
                         sem_sw[s])

    fire_idx(0, 0)
    fire_idx(1, 1)
    wait_idx(0)
    fire_gather(0)

    def loop_body(j, carry):
        wait_gather(0)

        @pl.when(2 * j + 2 <= ITERS - 1)
        def _():
            fire_idx(2 * j + 2, 0)

        @pl.when(2 * j + 1 <= ITERS - 1)
        def _():
            wait_idx(1)
            fire_gather(1)

        compute(2 * j, 0)

        @pl.when(2 * j + 1 <= ITERS - 1)
        def _():
            wait_gather(1)

            @pl.when(2 * j + 3 <= ITERS - 1)
            def _():
                fire_idx(2 * j + 3, 1)

            @pl.when(2 * j + 2 <= ITERS - 1)
            def _():
                wait_idx(0)
                fire_gather(0)

            compute(2 * j + 1, 1)

        return carry

    lax.fori_loop(0, (ITERS + 1) // 2, loop_body, 0)

    for s in range(2):
        pltpu.make_async_copy(ones, hist_sh.at[bbuf_r[s]], sem_sc[s]).wait()
        pltpu.make_async_copy(
            sbuf_r[s], scores_out.at[pl.ds(0, CHUNK)], sem_sw[s]).wait()

    plsc.subcore_barrier()
    pltpu.sync_copy(hist_sh.at[pl.ds(sid * STRIPE, STRIPE)],
                    hist_out.at[cid, pl.ds(sid * STRIPE, STRIPE)])


_SC_SCORE_CACHE = []


def _sc_score_fn():
    if not _SC_SCORE_CACHE:
        _SC_SCORE_CACHE.append(_build_sc_score())
    return _SC_SCORE_CACHE[0]


def _build_sc_score():
  return pl.kernel(
    _sc_body,
    out_type=(
        jax.ShapeDtypeStruct((E_PAD,), jnp.float32),
        jax.ShapeDtypeStruct((NC, HISTLEN), jnp.float32),
    ),
    mesh=plsc.VectorSubcoreMesh(core_axis_name="c", subcore_axis_name="s",
                                num_cores=NC, num_subcores=NS),
    compiler_params=pltpu.CompilerParams(needs_layout_passes=False),
    scratch_types=[
        pltpu.VMEM((CHUNK,), jnp.int32),
        pltpu.VMEM((CHUNK,), jnp.int32),
        pltpu.VMEM((CHUNK,), jnp.int32),
        pltpu.VMEM((CHUNK,), jnp.int32),
        pltpu.VMEM((CHUNK, D), jnp.float32),
        pltpu.VMEM((CHUNK, D), jnp.float32),
        pltpu.VMEM((CHUNK, D), jnp.float32),
        pltpu.VMEM((CHUNK, D), jnp.float32),
        pltpu.VMEM((CHUNK,), jnp.float32),
        pltpu.VMEM((CHUNK,), jnp.float32),
        pltpu.VMEM((CHUNK,), jnp.int32),
        pltpu.VMEM((CHUNK,), jnp.int32),
        pltpu.VMEM((CHUNK,), jnp.float32),
        pltpu.VMEM_SHARED((HISTLEN + NBUCK,), jnp.float32),
    ] + [pltpu.SemaphoreType.DMA] * 10,
  )


R = 512
C = NBUCK // R


def _tc_body(scores_ref, hp_ref, hn_ref, out_ref):
    s = scores_ref[...]
    rows = lax.broadcasted_iota(jnp.int32, s.shape, 0)
    lbl = (rows < NPOS_CHUNKS).astype(jnp.float32)
    bce = (jnp.maximum(s, 0.0) - s * lbl
           + jnp.log1p(jnp.exp(-jnp.abs(s))))
    loss = jnp.sum(bce) * (1.0 / E_TOTAL)

    p = hp_ref[0] + hp_ref[1]
    n = hn_ref[0] + hn_ref[1]
    ii = lax.broadcasted_iota(jnp.int32, (R, C), 0)
    jj = lax.broadcasted_iota(jnp.int32, (R, C), 1)
    upper = (ii < jj).astype(jnp.float32)
    lower = (ii > jj).astype(jnp.float32)
    cum_in_row = jnp.dot(n, upper, preferred_element_type=jnp.float32)
    rowsum = jnp.sum(n, axis=1, keepdims=True)
    rowprev = jnp.dot(lower, rowsum, preferred_element_type=jnp.float32)
    cum_excl = cum_in_row + rowprev

    inv = 1.0 / 320000.0
    term1 = jnp.sum(p * (cum_excl * inv))
    term2 = 0.5 * inv * jnp.sum(p * n)
    auc = (term1 + term2) * inv

    ij = lax.broadcasted_iota(jnp.int32, (8, 128), 1)
    i0 = lax.broadcasted_iota(jnp.int32, (8, 128), 0)
    out = jnp.where((i0 == 0) & (ij == 0), loss,
                    jnp.where((i0 == 0) & (ij == 1), auc, 0.0))
    out_ref[...] = out


def kernel(embeddings, pos_edges, neg_edges):
    pad = jnp.zeros((E_PAD - E_TOTAL,), jnp.int32)
    srcs = jnp.concatenate(
        [pos_edges[0].astype(jnp.int32), neg_edges[0].astype(jnp.int32), pad])
    dsts = jnp.concatenate(
        [pos_edges[1].astype(jnp.int32), neg_edges[1].astype(jnp.int32), pad])
    zeros = jnp.zeros((STRIPE,), jnp.float32)
    scores, hist = _sc_score_fn()(embeddings, srcs, dsts, zeros)

    scores2d = lax.slice(scores, (0,), (E_TOTAL,)).reshape(NCHUNKS, CHUNK)
    hp = hist[:, :NBUCK].reshape(NC, R, C)
    hn = hist[:, NBUCK:].reshape(NC, R, C)
    out = pl.pallas_call(
        _tc_body,
        out_shape=jax.ShapeDtypeStruct((8, 128), jnp.float32),
    )(scores2d, hp, hn)
    return out[0, 0], out[0, 1]


# speedup vs baseline: 4.4128x; 1.0487x over previous
import jax
import jax.numpy as jnp
from jax import lax
from jax.experimental import pallas as pl
from jax.experimental.pallas import tpu as pltpu
from jax.experimental.pallas import tpu_sc as plsc

D = 128
E_TOTAL = 640000
CHUNK = 128
NCHUNKS = E_TOTAL // CHUNK
NPOS_CHUNKS = 320000 // CHUNK
NC, NS = 2, 16
NW = NC * NS
ITERS = -(-NCHUNKS // NW)
NCHUNKS_PAD = NW * ITERS
E_PAD = NCHUNKS_PAD * CHUNK
HBITS = 18
NBUCK = 1 << HBITS
HISTLEN = 2 * NBUCK
STRIPE = HISTLEN // NS


def _sc_body(emb, srcs, dsts, zeros, scores_out, hist_out,
             idx_u0, idx_v0, idx_u1, idx_v1,
             rows_u0, rows_v0, rows_u1, rows_v1,
             sbuf0, sbuf1, bbuf0, bbuf1, ones, hist_sh,
             sem_i0, sem_i1, sem_gu0, sem_gv0, sem_gu1, sem_gv1,
             sem_sc0, sem_sc1, sem_sw0, sem_sw1):
    cid = lax.axis_index("c")
    sid = lax.axis_index("s")
    wid = sid * NC + cid

    idx_u = [idx_u0, idx_u1]
    idx_v = [idx_v0, idx_v1]
    rows_u = [rows_u0, rows_u1]
    rows_v = [rows_v0, rows_v1]
    sbuf_r = [sbuf0, sbuf1]
    bbuf_r = [bbuf0, bbuf1]
    sem_i = [sem_i0, sem_i1]
    sem_gu = [sem_gu0, sem_gu1]
    sem_gv = [sem_gv0, sem_gv1]
    sem_sc = [sem_sc0, sem_sc1]
    sem_sw = [sem_sw0, sem_sw1]

    pltpu.sync_copy(zeros, hist_sh.at[pl.ds(sid * STRIPE, STRIPE)])
    for j in range(CHUNK // 16):
        ones[pl.ds(j * 16, 16)] = jnp.full((16,), 1.0, jnp.float32)
    plsc.subcore_barrier()

    def cix(i):
        return wid + NW * i

    def fire_idx(i, s):
        base = cix(i) * CHUNK
        pltpu.async_copy(srcs.at[pl.ds(base, CHUNK)], idx_u[s], sem_i[s])
        pltpu.async_copy(dsts.at[pl.ds(base, CHUNK)], idx_v[s], sem_i[s])

    def wait_idx(s):
        pltpu.make_async_copy(srcs.at[pl.ds(0, CHUNK)], idx_u[s], sem_i[s]).wait()
        pltpu.make_async_copy(dsts.at[pl.ds(0, CHUNK)], idx_v[s], sem_i[s]).wait()

    def fire_gather(s):
        pltpu.async_copy(emb.at[idx_u[s]], rows_u[s], sem_gu[s])
        pltpu.async_copy(emb.at[idx_v[s]], rows_v[s], sem_gv[s])

    def wait_gather(s):
        pltpu.make_async_copy(emb.at[idx_u[s]], rows_u[s], sem_gu[s]).wait()
        pltpu.make_async_copy(emb.at[idx_v[s]], rows_v[s], sem_gv[s]).wait()

    lane = lax.iota(jnp.int32, 16)

    def compute(i, s):
        c = cix(i)
        ru, rv = rows_u[s], rows_v[s]
        sbuf, bbuf = sbuf_r[s], bbuf_r[s]

        @pl.when(i >= 2)
        def _():
            pltpu.make_async_copy(ones, hist_sh.at[bbuf], sem_sc[s]).wait()
            pltpu.make_async_copy(
                sbuf, scores_out.at[pl.ds(0, CHUNK)], sem_sw[s]).wait()
        def group_body(g, carry2):
            rows16 = g * 16 + lane

            z = jnp.zeros((16,), jnp.float32)

            @plsc.parallel_loop(0, D // 4, unroll=8, carry=(z, z, z, z))
            def accs(db, accs_in):
                new = []
                for k in range(4):
                    col = (lane + (db * 4 + k)) & jnp.int32(D - 1)
                    cu = plsc.load_gather(ru, [rows16, col])
                    cv = plsc.load_gather(rv, [rows16, col])
                    new.append(accs_in[k] + cu * cv)
                return tuple(new)
            vec = (accs[0] + accs[1]) + (accs[2] + accs[3])
            sbuf[pl.ds(g * 16, 16)] = vec
            return carry2

        lax.fori_loop(0, CHUNK // 16, group_body, 0)

        off = (jnp.where(c < NPOS_CHUNKS, 0, NBUCK)
               + jnp.where(c < NCHUNKS, 0, NBUCK)).astype(jnp.int32)
        for j in range(CHUNK // 16):
            sv = sbuf[pl.ds(j * 16, 16)]
            b = plsc.bitcast(sv, jnp.int32)
            key = jnp.where(b < 0, b ^ jnp.int32(0x7FFFFFFF), b)
            bkt = lax.shift_right_arithmetic(key, 32 - HBITS)
            bkt = bkt + jnp.int32(NBUCK // 2) + off
            bbuf[pl.ds(j * 16, 16)] = bkt

        pltpu.async_copy(ones, hist_sh.at[bbuf], sem_sc[s], add=True)
        pltpu.async_copy(sbuf, scores_out.at[pl.ds(c * CHUNK, CHUNK)],
                         sem_sw[s])

    fire_idx(0, 0)
    fire_idx(1, 1)
    wait_idx(0)
    fire_gather(0)

    def loop_body(j, carry):
        @pl.when(2 * j + 1 <= ITERS - 1)
        def _():
            wait_idx(1)
            fire_gather(1)

        wait_gather(0)

        @pl.when(2 * j + 2 <= ITERS - 1)
        def _():
            fire_idx(2 * j + 2, 0)

        compute(2 * j, 0)

        @pl.when(2 * j + 1 <= ITERS - 1)
        def _():
            @pl.when(2 * j + 2 <= ITERS - 1)
            def _():
                wait_idx(0)
                fire_gather(0)

            wait_gather(1)

            @pl.when(2 * j + 3 <= ITERS - 1)
            def _():
                fire_idx(2 * j + 3, 1)

            compute(2 * j + 1, 1)

        return carry

    lax.fori_loop(0, (ITERS + 1) // 2, loop_body, 0)

    for s in range(2):
        pltpu.make_async_copy(ones, hist_sh.at[bbuf_r[s]], sem_sc[s]).wait()
        pltpu.make_async_copy(
            sbuf_r[s], scores_out.at[pl.ds(0, CHUNK)], sem_sw[s]).wait()

    plsc.subcore_barrier()
    pltpu.sync_copy(hist_sh.at[pl.ds(sid * STRIPE, STRIPE)],
                    hist_out.at[cid, pl.ds(sid * STRIPE, STRIPE)])


_SC_SCORE_CACHE = []


def _sc_score_fn():
    if not _SC_SCORE_CACHE:
        _SC_SCORE_CACHE.append(_build_sc_score())
    return _SC_SCORE_CACHE[0]


def _build_sc_score():
  return pl.kernel(
    _sc_body,
    out_type=(
        jax.ShapeDtypeStruct((E_PAD,), jnp.float32),
        jax.ShapeDtypeStruct((NC, HISTLEN), jnp.float32),
    ),
    mesh=plsc.VectorSubcoreMesh(core_axis_name="c", subcore_axis_name="s",
                                num_cores=NC, num_subcores=NS),
    compiler_params=pltpu.CompilerParams(needs_layout_passes=False),
    scratch_types=[
        pltpu.VMEM((CHUNK,), jnp.int32),
        pltpu.VMEM((CHUNK,), jnp.int32),
        pltpu.VMEM((CHUNK,), jnp.int32),
        pltpu.VMEM((CHUNK,), jnp.int32),
        pltpu.VMEM((CHUNK, D), jnp.float32),
        pltpu.VMEM((CHUNK, D), jnp.float32),
        pltpu.VMEM((CHUNK, D), jnp.float32),
        pltpu.VMEM((CHUNK, D), jnp.float32),
        pltpu.VMEM((CHUNK,), jnp.float32),
        pltpu.VMEM((CHUNK,), jnp.float32),
        pltpu.VMEM((CHUNK,), jnp.int32),
        pltpu.VMEM((CHUNK,), jnp.int32),
        pltpu.VMEM((CHUNK,), jnp.float32),
        pltpu.VMEM_SHARED((HISTLEN + NBUCK,), jnp.float32),
    ] + [pltpu.SemaphoreType.DMA] * 10,
  )


R = 512
C = NBUCK // R


def _tc_body(scores_ref, hp_ref, hn_ref, out_ref):
    s = scores_ref[...]
    rows = lax.broadcasted_iota(jnp.int32, s.shape, 0)
    lbl = (rows < NPOS_CHUNKS).astype(jnp.float32)
    bce = (jnp.maximum(s, 0.0) - s * lbl
           + jnp.log1p(jnp.exp(-jnp.abs(s))))
    loss = jnp.sum(bce) * (1.0 / E_TOTAL)

    p = hp_ref[0] + hp_ref[1]
    n = hn_ref[0] + hn_ref[1]
    ii = lax.broadcasted_iota(jnp.int32, (R, C), 0)
    jj = lax.broadcasted_iota(jnp.int32, (R, C), 1)
    upper = (ii < jj).astype(jnp.float32)
    lower = (ii > jj).astype(jnp.float32)
    cum_in_row = jnp.dot(n, upper, preferred_element_type=jnp.float32)
    rowsum = jnp.sum(n, axis=1, keepdims=True)
    rowprev = jnp.dot(lower, rowsum, preferred_element_type=jnp.float32)
    cum_excl = cum_in_row + rowprev

    inv = 1.0 / 320000.0
    term1 = jnp.sum(p * (cum_excl * inv))
    term2 = 0.5 * inv * jnp.sum(p * n)
    auc = (term1 + term2) * inv

    ij = lax.broadcasted_iota(jnp.int32, (8, 128), 1)
    i0 = lax.broadcasted_iota(jnp.int32, (8, 128), 0)
    out = jnp.where((i0 == 0) & (ij == 0), loss,
                    jnp.where((i0 == 0) & (ij == 1), auc, 0.0))
    out_ref[...] = out


def kernel(embeddings, pos_edges, neg_edges):
    pad = jnp.zeros((E_PAD - E_TOTAL,), jnp.int32)
    srcs = jnp.concatenate(
        [pos_edges[0].astype(jnp.int32), neg_edges[0].astype(jnp.int32), pad])
    dsts = jnp.concatenate(
        [pos_edges[1].astype(jnp.int32), neg_edges[1].astype(jnp.int32), pad])
    zeros = jnp.zeros((STRIPE,), jnp.float32)
    scores, hist = _sc_score_fn()(embeddings, srcs, dsts, zeros)

    scores2d = lax.slice(scores, (0,), (E_TOTAL,)).reshape(NCHUNKS, CHUNK)
    hp = hist[:, :NBUCK].reshape(NC, R, C)
    hn = hist[:, NBUCK:].reshape(NC, R, C)
    out = pl.pallas_call(
        _tc_body,
        out_shape=jax.ShapeDtypeStruct((8, 128), jnp.float32),
    )(scores2d, hp, hn)
    return out[0, 0], out[0, 1]


# speedup vs baseline: 7.7311x; 1.7520x over previous
import jax
import jax.numpy as jnp
from jax import lax
from jax.experimental import pallas as pl
from jax.experimental.pallas import tpu as pltpu
from jax.experimental.pallas import tpu_sc as plsc

D = 128
E_TOTAL = 640000
CHUNK = 128
NCHUNKS = E_TOTAL // CHUNK
NPOS_CHUNKS = 320000 // CHUNK
NC, NS = 2, 16
NW = NC * NS
ITERS = -(-NCHUNKS // NW)
NCHUNKS_PAD = NW * ITERS
E_PAD = NCHUNKS_PAD * CHUNK
HBITS = 18
NBUCK = 1 << HBITS
HISTLEN = 2 * NBUCK
STRIPE = HISTLEN // NS


def _sc_body(emb, pos, neg, zeros, scores_out, hist_out,
             idx_u0, idx_v0, idx_u1, idx_v1,
             rows_u0, rows_v0, rows_u1, rows_v1,
             sbuf0, sbuf1, bbuf0, bbuf1, ones, hist_sh,
             sem_i0, sem_i1, sem_gu0, sem_gv0, sem_gu1, sem_gv1,
             sem_sc0, sem_sc1, sem_sw0, sem_sw1):
    cid = lax.axis_index("c")
    sid = lax.axis_index("s")
    wid = sid * NC + cid

    idx_u = [idx_u0, idx_u1]
    idx_v = [idx_v0, idx_v1]
    rows_u = [rows_u0, rows_u1]
    rows_v = [rows_v0, rows_v1]
    sbuf_r = [sbuf0, sbuf1]
    bbuf_r = [bbuf0, bbuf1]
    sem_i = [sem_i0, sem_i1]
    sem_gu = [sem_gu0, sem_gu1]
    sem_gv = [sem_gv0, sem_gv1]
    sem_sc = [sem_sc0, sem_sc1]
    sem_sw = [sem_sw0, sem_sw1]

    pltpu.sync_copy(zeros, hist_sh.at[pl.ds(sid * STRIPE, STRIPE)])
    for j in range(CHUNK // 16):
        ones[pl.ds(j * 16, 16)] = jnp.full((16,), 1.0, jnp.float32)
    plsc.subcore_barrier()

    def cix(i):
        return wid + NW * i

    def fire_idx(i, s):
        c = cix(i)
        base = c * CHUNK

        @pl.when(c < NPOS_CHUNKS)
        def _():
            pltpu.async_copy(pos.at[0, pl.ds(base, CHUNK)], idx_u[s], sem_i[s])
            pltpu.async_copy(pos.at[1, pl.ds(base, CHUNK)], idx_v[s], sem_i[s])

        @pl.when((c >= NPOS_CHUNKS) & (c < NCHUNKS))
        def _():
            nbase = base - NPOS_CHUNKS * CHUNK
            pltpu.async_copy(neg.at[0, pl.ds(nbase, CHUNK)], idx_u[s], sem_i[s])
            pltpu.async_copy(neg.at[1, pl.ds(nbase, CHUNK)], idx_v[s], sem_i[s])

        @pl.when(c >= NCHUNKS)
        def _():
            pltpu.async_copy(neg.at[0, pl.ds(0, CHUNK)], idx_u[s], sem_i[s])
            pltpu.async_copy(neg.at[1, pl.ds(0, CHUNK)], idx_v[s], sem_i[s])

    def wait_idx(s):
        pltpu.make_async_copy(pos.at[0, pl.ds(0, CHUNK)], idx_u[s], sem_i[s]).wait()
        pltpu.make_async_copy(pos.at[1, pl.ds(0, CHUNK)], idx_v[s], sem_i[s]).wait()

    def fire_gather(s):
        pltpu.async_copy(emb.at[idx_u[s]], rows_u[s], sem_gu[s])
        pltpu.async_copy(emb.at[idx_v[s]], rows_v[s], sem_gv[s])

    def wait_gather(s):
        pltpu.make_async_copy(emb.at[idx_u[s]], rows_u[s], sem_gu[s]).wait()
        pltpu.make_async_copy(emb.at[idx_v[s]], rows_v[s], sem_gv[s]).wait()

    lane = lax.iota(jnp.int32, 16)

    def compute(i, s):
        c = cix(i)
        ru, rv = rows_u[s], rows_v[s]
        sbuf, bbuf = sbuf_r[s], bbuf_r[s]

        @pl.when(i >= 2)
        def _():
            pltpu.make_async_copy(ones, hist_sh.at[bbuf], sem_sc[s]).wait()
            pltpu.make_async_copy(
                sbuf, scores_out.at[pl.ds(0, CHUNK)], sem_sw[s]).wait()
        def group_body(g, carry2):
            rows16 = g * 16 + lane

            z = jnp.zeros((16,), jnp.float32)

            @plsc.parallel_loop(0, D // 4, unroll=8, carry=(z, z, z, z))
            def accs(db, accs_in):
                new = []
                for k in range(4):
                    col = (lane + (db * 4 + k)) & jnp.int32(D - 1)
                    cu = plsc.load_gather(ru, [rows16, col])
                    cv = plsc.load_gather(rv, [rows16, col])
                    new.append(accs_in[k] + cu * cv)
                return tuple(new)
            vec = (accs[0] + accs[1]) + (accs[2] + accs[3])
            sbuf[pl.ds(g * 16, 16)] = vec
            return carry2

        lax.fori_loop(0, CHUNK // 16, group_body, 0)

        off = (jnp.where(c < NPOS_CHUNKS, 0, NBUCK)
               + jnp.where(c < NCHUNKS, 0, NBUCK)).astype(jnp.int32)
        for j in range(CHUNK // 16):
            sv = sbuf[pl.ds(j * 16, 16)]
            b = plsc.bitcast(sv, jnp.int32)
            key = jnp.where(b < 0, b ^ jnp.int32(0x7FFFFFFF), b)
            bkt = lax.shift_right_arithmetic(key, 32 - HBITS)
            bkt = bkt + jnp.int32(NBUCK // 2) + off
            bbuf[pl.ds(j * 16, 16)] = bkt

        pltpu.async_copy(ones, hist_sh.at[bbuf], sem_sc[s], add=True)
        pltpu.async_copy(sbuf, scores_out.at[pl.ds(c * CHUNK, CHUNK)],
                         sem_sw[s])

    fire_idx(0, 0)
    fire_idx(1, 1)
    wait_idx(0)
    fire_gather(0)

    def loop_body(j, carry):
        @pl.when(2 * j + 1 <= ITERS - 1)
        def _():
            wait_idx(1)
            fire_gather(1)

        wait_gather(0)

        @pl.when(2 * j + 2 <= ITERS - 1)
        def _():
            fire_idx(2 * j + 2, 0)

        compute(2 * j, 0)

        @pl.when(2 * j + 1 <= ITERS - 1)
        def _():
            @pl.when(2 * j + 2 <= ITERS - 1)
            def _():
                wait_idx(0)
                fire_gather(0)

            wait_gather(1)

            @pl.when(2 * j + 3 <= ITERS - 1)
            def _():
                fire_idx(2 * j + 3, 1)

            compute(2 * j + 1, 1)

        return carry

    lax.fori_loop(0, (ITERS + 1) // 2, loop_body, 0)

    for s in range(2):
        pltpu.make_async_copy(ones, hist_sh.at[bbuf_r[s]], sem_sc[s]).wait()
        pltpu.make_async_copy(
            sbuf_r[s], scores_out.at[pl.ds(0, CHUNK)], sem_sw[s]).wait()

    plsc.subcore_barrier()
    pltpu.sync_copy(hist_sh.at[pl.ds(sid * STRIPE, STRIPE)],
                    hist_out.at[cid, pl.ds(sid * STRIPE, STRIPE)])


_SC_SCORE_CACHE = []


def _sc_score_fn():
    if not _SC_SCORE_CACHE:
        _SC_SCORE_CACHE.append(_build_sc_score())
    return _SC_SCORE_CACHE[0]


def _build_sc_score():
  return pl.kernel(
    _sc_body,
    out_type=(
        jax.ShapeDtypeStruct((E_PAD,), jnp.float32),
        jax.ShapeDtypeStruct((NC, HISTLEN), jnp.float32),
    ),
    mesh=plsc.VectorSubcoreMesh(core_axis_name="c", subcore_axis_name="s",
                                num_cores=NC, num_subcores=NS),
    compiler_params=pltpu.CompilerParams(needs_layout_passes=False),
    scratch_types=[
        pltpu.VMEM((CHUNK,), jnp.int32),
        pltpu.VMEM((CHUNK,), jnp.int32),
        pltpu.VMEM((CHUNK,), jnp.int32),
        pltpu.VMEM((CHUNK,), jnp.int32),
        pltpu.VMEM((CHUNK, D), jnp.float32),
        pltpu.VMEM((CHUNK, D), jnp.float32),
        pltpu.VMEM((CHUNK, D), jnp.float32),
        pltpu.VMEM((CHUNK, D), jnp.float32),
        pltpu.VMEM((CHUNK,), jnp.float32),
        pltpu.VMEM((CHUNK,), jnp.float32),
        pltpu.VMEM((CHUNK,), jnp.int32),
        pltpu.VMEM((CHUNK,), jnp.int32),
        pltpu.VMEM((CHUNK,), jnp.float32),
        pltpu.VMEM_SHARED((HISTLEN + NBUCK,), jnp.float32),
    ] + [pltpu.SemaphoreType.DMA] * 10,
  )


R = 512
C = NBUCK // R


def _tc_body(scores_ref, hp_ref, hn_ref, out_ref):
    s = scores_ref[...]
    rows = lax.broadcasted_iota(jnp.int32, s.shape, 0)
    lbl = (rows < NPOS_CHUNKS).astype(jnp.float32)
    valid = (rows < NCHUNKS).astype(jnp.float32)
    bce = (jnp.maximum(s, 0.0) - s * lbl
           + jnp.log1p(jnp.exp(-jnp.abs(s)))) * valid
    loss = jnp.sum(bce) * (1.0 / E_TOTAL)

    p = hp_ref[0] + hp_ref[1]
    n = hn_ref[0] + hn_ref[1]
    ii = lax.broadcasted_iota(jnp.int32, (R, C), 0)
    jj = lax.broadcasted_iota(jnp.int32, (R, C), 1)
    upper = (ii < jj).astype(jnp.float32)
    lower = (ii > jj).astype(jnp.float32)
    cum_in_row = jnp.dot(n, upper, preferred_element_type=jnp.float32)
    rowsum = jnp.sum(n, axis=1, keepdims=True)
    rowprev = jnp.dot(lower, rowsum, preferred_element_type=jnp.float32)
    cum_excl = cum_in_row + rowprev

    inv = 1.0 / 320000.0
    term1 = jnp.sum(p * (cum_excl * inv))
    term2 = 0.5 * inv * jnp.sum(p * n)
    auc = (term1 + term2) * inv

    ij = lax.broadcasted_iota(jnp.int32, (8, 128), 1)
    i0 = lax.broadcasted_iota(jnp.int32, (8, 128), 0)
    out = jnp.where((i0 == 0) & (ij == 0), loss,
                    jnp.where((i0 == 0) & (ij == 1), auc, 0.0))
    out_ref[...] = out


def kernel(embeddings, pos_edges, neg_edges):
    zeros = jnp.zeros((STRIPE,), jnp.float32)
    scores, hist = _sc_score_fn()(
        embeddings, pos_edges.astype(jnp.int32), neg_edges.astype(jnp.int32),
        zeros)

    scores2d = scores.reshape(NCHUNKS_PAD, CHUNK)
    hp = hist[:, :NBUCK].reshape(NC, R, C)
    hn = hist[:, NBUCK:].reshape(NC, R, C)
    out = pl.pallas_call(
        _tc_body,
        out_shape=jax.ShapeDtypeStruct((8, 128), jnp.float32),
    )(scores2d, hp, hn)
    return out[0, 0], out[0, 1]


# speedup vs baseline: 9.1439x; 1.1827x over previous
import jax
import jax.numpy as jnp
from jax import lax
from jax.experimental import pallas as pl
from jax.experimental.pallas import tpu as pltpu
from jax.experimental.pallas import tpu_sc as plsc

D = 128
E_TOTAL = 640000
CHUNK = 128
NCHUNKS = E_TOTAL // CHUNK
NPOS_CHUNKS = 320000 // CHUNK
NC, NS = 2, 16
NW = NC * NS
ITERS = -(-NCHUNKS // NW)
NCHUNKS_PAD = NW * ITERS
E_PAD = NCHUNKS_PAD * CHUNK
HBITS = 17
NBUCK = 1 << HBITS
HISTLEN = 2 * NBUCK
STRIPE = HISTLEN // NS


NBUF = 3


def _sc_body(emb, pos, neg, zeros, scores_out, hist_out,
             idx_u0, idx_v0, idx_u1, idx_v1, idx_u2, idx_v2,
             rows_u0, rows_v0, rows_u1, rows_v1, rows_u2, rows_v2,
             sbuf0, sbuf1, sbuf2, bbuf0, bbuf1, bbuf2, ones, hist_sh,
             sem_i0, sem_i1, sem_i2,
             sem_gu0, sem_gv0, sem_gu1, sem_gv1, sem_gu2, sem_gv2,
             sem_sc0, sem_sc1, sem_sc2, sem_sw0, sem_sw1, sem_sw2):
    cid = lax.axis_index("c")
    sid = lax.axis_index("s")
    wid = sid * NC + cid

    idx_u = [idx_u0, idx_u1, idx_u2]
    idx_v = [idx_v0, idx_v1, idx_v2]
    rows_u = [rows_u0, rows_u1, rows_u2]
    rows_v = [rows_v0, rows_v1, rows_v2]
    sbuf_r = [sbuf0, sbuf1, sbuf2]
    bbuf_r = [bbuf0, bbuf1, bbuf2]
    sem_i = [sem_i0, sem_i1, sem_i2]
    sem_gu = [sem_gu0, sem_gu1, sem_gu2]
    sem_gv = [sem_gv0, sem_gv1, sem_gv2]
    sem_sc = [sem_sc0, sem_sc1, sem_sc2]
    sem_sw = [sem_sw0, sem_sw1, sem_sw2]

    pltpu.sync_copy(zeros, hist_sh.at[pl.ds(sid * STRIPE, STRIPE)])
    for j in range(CHUNK // 16):
        ones[pl.ds(j * 16, 16)] = jnp.full((16,), 1.0, jnp.float32)
    plsc.subcore_barrier()

    def cix(i):
        return wid + NW * i

    def fire_idx(i, s):
        c = cix(i)
        base = c * CHUNK

        @pl.when(c < NPOS_CHUNKS)
        def _():
            pltpu.async_copy(pos.at[0, pl.ds(base, CHUNK)], idx_u[s], sem_i[s])
            pltpu.async_copy(pos.at[1, pl.ds(base, CHUNK)], idx_v[s], sem_i[s])

        @pl.when((c >= NPOS_CHUNKS) & (c < NCHUNKS))
        def _():
            nbase = base - NPOS_CHUNKS * CHUNK
            pltpu.async_copy(neg.at[0, pl.ds(nbase, CHUNK)], idx_u[s], sem_i[s])
            pltpu.async_copy(neg.at[1, pl.ds(nbase, CHUNK)], idx_v[s], sem_i[s])

        @pl.when(c >= NCHUNKS)
        def _():
            pltpu.async_copy(neg.at[0, pl.ds(0, CHUNK)], idx_u[s], sem_i[s])
            pltpu.async_copy(neg.at[1, pl.ds(0, CHUNK)], idx_v[s], sem_i[s])

    def wait_idx(s):
        pltpu.make_async_copy(pos.at[0, pl.ds(0, CHUNK)], idx_u[s], sem_i[s]).wait()
        pltpu.make_async_copy(pos.at[1, pl.ds(0, CHUNK)], idx_v[s], sem_i[s]).wait()

    def fire_gather(s):
        pltpu.async_copy(emb.at[idx_u[s]], rows_u[s], sem_gu[s])
        pltpu.async_copy(emb.at[idx_v[s]], rows_v[s], sem_gv[s])

    def wait_gather(s):
        pltpu.make_async_copy(emb.at[idx_u[s]], rows_u[s], sem_gu[s]).wait()
        pltpu.make_async_copy(emb.at[idx_v[s]], rows_v[s], sem_gv[s]).wait()

    lane = lax.iota(jnp.int32, 16)

    def compute(i, s):
        c = cix(i)
        ru, rv = rows_u[s], rows_v[s]
        sbuf, bbuf = sbuf_r[s], bbuf_r[s]

        @pl.when(i >= NBUF)
        def _():
            pltpu.make_async_copy(ones, hist_sh.at[bbuf], sem_sc[s]).wait()
            pltpu.make_async_copy(
                sbuf, scores_out.at[pl.ds(0, CHUNK)], sem_sw[s]).wait()
        def group_body(g, carry2):
            rows16 = g * 16 + lane

            z = jnp.zeros((16,), jnp.float32)

            @plsc.parallel_loop(0, D // 4, unroll=8, carry=(z, z, z, z))
            def accs(db, accs_in):
                new = []
                for k in range(4):
                    col = (lane + (db * 4 + k)) & jnp.int32(D - 1)
                    cu = plsc.load_gather(ru, [rows16, col])
                    cv = plsc.load_gather(rv, [rows16, col])
                    new.append(accs_in[k] + cu * cv)
                return tuple(new)
            vec = (accs[0] + accs[1]) + (accs[2] + accs[3])
            sbuf[pl.ds(g * 16, 16)] = vec
            return carry2

        lax.fori_loop(0, CHUNK // 16, group_body, 0)

        off = jnp.where(c < NPOS_CHUNKS, 0, NBUCK).astype(jnp.int32)
        is_pad = c >= NCHUNKS
        for j in range(CHUNK // 16):
            sv = sbuf[pl.ds(j * 16, 16)]
            b = plsc.bitcast(sv, jnp.int32)
            key = jnp.where(b < 0, b ^ jnp.int32(0x7FFFFFFF), b)
            bkt = lax.shift_right_arithmetic(key, 32 - HBITS)
            bkt = bkt + jnp.int32(NBUCK // 2)
            bkt = jnp.where(is_pad, HISTLEN + (bkt & jnp.int32(127)),
                            bkt + off)
            bbuf[pl.ds(j * 16, 16)] = bkt

        pltpu.async_copy(ones, hist_sh.at[bbuf], sem_sc[s], add=True)
        pltpu.async_copy(sbuf, scores_out.at[pl.ds(c * CHUNK, CHUNK)],
                         sem_sw[s])

    for s in range(NBUF):
        fire_idx(s, s)
    wait_idx(0)
    fire_gather(0)
    wait_idx(1)
    fire_gather(1)

    def loop_body(j, carry):
        for t in range(NBUF):
            i = NBUF * j + t

            @pl.when(i <= ITERS - 1)
            def _(i=i, t=t):
                @pl.when(i + 2 <= ITERS - 1)
                def _():
                    wait_idx((t + 2) % NBUF)
                    fire_gather((t + 2) % NBUF)

                wait_gather(t)

                @pl.when(i + NBUF <= ITERS - 1)
                def _():
                    fire_idx(i + NBUF, t)

                compute(i, t)

        return carry

    lax.fori_loop(0, -(-ITERS // NBUF), loop_body, 0)

    for s in range(NBUF):
        pltpu.make_async_copy(ones, hist_sh.at[bbuf_r[s]], sem_sc[s]).wait()
        pltpu.make_async_copy(
            sbuf_r[s], scores_out.at[pl.ds(0, CHUNK)], sem_sw[s]).wait()

    plsc.subcore_barrier()
    pltpu.sync_copy(hist_sh.at[pl.ds(sid * STRIPE, STRIPE)],
                    hist_out.at[cid, pl.ds(sid * STRIPE, STRIPE)])


_SC_SCORE_CACHE = []


def _sc_score_fn():
    if not _SC_SCORE_CACHE:
        _SC_SCORE_CACHE.append(_build_sc_score())
    return _SC_SCORE_CACHE[0]


def _build_sc_score():
  return pl.kernel(
    _sc_body,
    out_type=(
        jax.ShapeDtypeStruct((E_PAD,), jnp.float32),
        jax.ShapeDtypeStruct((NC, HISTLEN), jnp.float32),
    ),
    mesh=plsc.VectorSubcoreMesh(core_axis_name="c", subcore_axis_name="s",
                                num_cores=NC, num_subcores=NS),
    compiler_params=pltpu.CompilerParams(needs_layout_passes=False),
    scratch_types=(
        [pltpu.VMEM((CHUNK,), jnp.int32)] * (2 * NBUF)
        + [pltpu.VMEM((CHUNK, D), jnp.float32)] * (2 * NBUF)
        + [pltpu.VMEM((CHUNK,), jnp.float32)] * NBUF
        + [pltpu.VMEM((CHUNK,), jnp.int32)] * NBUF
        + [pltpu.VMEM((CHUNK,), jnp.float32)]
        + [pltpu.VMEM_SHARED((HISTLEN + 128,), jnp.float32)]
        + [pltpu.SemaphoreType.DMA] * (5 * NBUF)
    ),
  )


R = 512
C = NBUCK // R


def _tc_body(scores_ref, hp_ref, hn_ref, out_ref):
    s = scores_ref[...]
    rows = lax.broadcasted_iota(jnp.int32, s.shape, 0)
    lbl = (rows < NPOS_CHUNKS).astype(jnp.float32)
    valid = (rows < NCHUNKS).astype(jnp.float32)
    bce = (jnp.maximum(s, 0.0) - s * lbl
           + jnp.log1p(jnp.exp(-jnp.abs(s)))) * valid
    loss = jnp.sum(bce) * (1.0 / E_TOTAL)

    p = hp_ref[0] + hp_ref[1]
    n = hn_ref[0] + hn_ref[1]
    ic = lax.broadcasted_iota(jnp.int32, (C, C), 0)
    jc = lax.broadcasted_iota(jnp.int32, (C, C), 1)
    upper = (ic < jc).astype(jnp.float32)
    ir = lax.broadcasted_iota(jnp.int32, (R, R), 0)
    jr = lax.broadcasted_iota(jnp.int32, (R, R), 1)
    lower = (ir > jr).astype(jnp.float32)
    cum_in_row = jnp.dot(n, upper, preferred_element_type=jnp.float32)
    rowsum = jnp.sum(n, axis=1, keepdims=True)
    rowprev = jnp.dot(lower, rowsum, preferred_element_type=jnp.float32)
    cum_excl = cum_in_row + rowprev

    inv = 1.0 / 320000.0
    term1 = jnp.sum(p * (cum_excl * inv))
    term2 = 0.5 * inv * jnp.sum(p * n)
    auc = (term1 + term2) * inv

    ij = lax.broadcasted_iota(jnp.int32, (8, 128), 1)
    i0 = lax.broadcasted_iota(jnp.int32, (8, 128), 0)
    out = jnp.where((i0 == 0) & (ij == 0), loss,
                    jnp.where((i0 == 0) & (ij == 1), auc, 0.0))
    out_ref[...] = out


def kernel(embeddings, pos_edges, neg_edges):
    zeros = jnp.zeros((STRIPE,), jnp.float32)
    scores, hist = _sc_score_fn()(
        embeddings, pos_edges.astype(jnp.int32), neg_edges.astype(jnp.int32),
        zeros)

    scores2d = scores.reshape(NCHUNKS_PAD, CHUNK)
    hp = hist[:, :NBUCK].reshape(NC, R, C)
    hn = hist[:, NBUCK:].reshape(NC, R, C)
    out = pl.pallas_call(
        _tc_body,
        out_shape=jax.ShapeDtypeStruct((8, 128), jnp.float32),
    )(scores2d, hp, hn)
    return out[0, 0], out[0, 1]
